# baseline (speedup 1.0000x reference)
import functools

import jax
import jax.numpy as jnp
from jax.experimental import pallas as pl
from jax.experimental.pallas import tpu as pltpu

_VMEM = pltpu.CompilerParams(vmem_limit_bytes=100 * 1024 * 1024)


def _lrelu(v):
    return jnp.where(v >= 0, v, 0.2 * v)


def _main_body(x_ref, wu1_ref, wu2_ref, ba_ref, bb_ref, wa_ref, wb_ref,
               b2_ref, b4_ref, b6_ref, v3_ref, c3_ref, adj1_ref, adj2_ref,
               x1_ref, x2_ref, x1c_ref, x2c_ref, emb_ref, s_ref, sq_ref,
               buf0, buf1, u_s, t_s, z_s, sem0, sem1, sem0b, sem1b,
               *, bm, nsteps):
    n = x_ref.shape[0]
    h2 = wu1_ref.shape[1]
    d = b2_ref.shape[1]
    i = pl.program_id(0)
    total = 4 * nsteps

    q = (bm // 32) * 8
    parts = [q, q, q, bm - 3 * q]
    offs = [0, q, 2 * q, 3 * q]

    def _dma(k, bufref, sems, *, wait):
        pk = k // nsteps
        rk = (k % nsteps) * bm

        def act(src):
            cs = [pltpu.make_async_copy(
                      src.at[pl.ds(rk + o, pz), :],
                      bufref.at[pl.ds(o, pz), :], sm)
                  for o, pz, sm in zip(offs, parts, sems)]
            for c in cs:
                if wait:
                    c.wait()
                else:
                    c.start()

        @pl.when((pk == 0) | (pk == 2))
        def _():
            act(adj1_ref)

        @pl.when((pk == 1) | (pk == 3))
        def _():
            act(adj2_ref)

    def dma_start(k, bufref, semref, semref2):
        _dma(k, bufref, [semref.at[j] for j in range(2)]
             + [semref2.at[j] for j in range(2)], wait=False)

    def dma_wait(k, bufref, semref, semref2):
        _dma(k, bufref, [semref.at[j] for j in range(2)]
             + [semref2.at[j] for j in range(2)], wait=True)

    @pl.when(i == 0)
    def _():
        dma_start(0, buf0, sem0, sem0b)
        nchunk = 10 if n % 10 == 0 else 1
        cs = n // nchunk
        for c in range(nchunk):
            xc_v = x_ref[pl.ds(c * cs, cs), :]
            u_s[pl.ds(c * cs, cs), 0:h2] = jnp.dot(
                xc_v, wu1_ref[...], preferred_element_type=jnp.float32)
            u_s[pl.ds(c * cs, cs), h2:2 * h2] = jnp.dot(
                xc_v, wu2_ref[...], preferred_element_type=jnp.float32)

    nxt = i + 1

    @pl.when((nxt < total) & (nxt % 2 == 0))
    def _():
        dma_start(nxt, buf0, sem0, sem0b)

    @pl.when((nxt < total) & (nxt % 2 == 1))
    def _():
        dma_start(nxt, buf1, sem1, sem1b)

    p = i // nsteps
    r = (i % nsteps) * bm

    def compute(aref):
        @pl.when(p == 0)
        def _():
            y = jnp.dot(aref[...], u_s[:, 0:h2],
                        preferred_element_type=jnp.float32)
            t_s[pl.ds(r, bm), 0:2 * d] = jnp.dot(
                _lrelu(y + ba_ref[...]), wa_ref[...],
                preferred_element_type=jnp.float32)

        @pl.when(p == 1)
        def _():
            y = jnp.dot(aref[...], u_s[:, h2:2 * h2],
                        preferred_element_type=jnp.float32)
            t_s[pl.ds(r, bm), 2 * d:4 * d] = jnp.dot(
                _lrelu(y + bb_ref[...]), wb_ref[...],
                preferred_element_type=jnp.float32)

        @pl.when(p == 2)
        def _():
            z_s[pl.ds(r, bm), 0:2 * d] = jnp.dot(
                aref[...], t_s[:, 0:2 * d],
                preferred_element_type=jnp.float32)
            z_s[pl.ds(r, bm), 2 * d:3 * d] = jnp.dot(
                aref[...], t_s[:, 3 * d:4 * d],
                preferred_element_type=jnp.float32)

        @pl.when(p == 3)
        def _():
            z2 = jnp.dot(aref[...], t_s[:, 2 * d:3 * d],
                         preferred_element_type=jnp.float32)
            za = z_s[pl.ds(r, bm), 0:2 * d]
            x1 = za[:, 0:d] + b2_ref[...]
            x1c = za[:, d:2 * d] + b6_ref[...]
            x2c = z_s[pl.ds(r, bm), 2 * d:3 * d] + b6_ref[...]
            x2 = z2 + b4_ref[...]
            xc = (x1c + x2c) * 0.5

            x3 = jnp.concatenate([x1, x2, xc], axis=1)
            s = jnp.dot(x3, v3_ref[...], preferred_element_type=jnp.float32)
            s = _lrelu(s + c3_ref[...])
            m = jnp.max(s, axis=1, keepdims=True)
            e = jnp.exp(s - m)
            w = e / jnp.sum(e, axis=1, keepdims=True)
            emb = w[:, 0:1] * x1 + w[:, 1:2] * x2 + w[:, 2:3] * xc

            x1_ref[...] = x1
            x2_ref[...] = x2
            x1c_ref[...] = x1c
            x2c_ref[...] = x2c
            emb_ref[...] = emb
            ps = jnp.sum(emb, axis=0, keepdims=True)
            psq = jnp.sum(emb * emb, axis=0, keepdims=True)

            @pl.when(i == 3 * nsteps)
            def _():
                s_ref[...] = ps
                sq_ref[...] = psq

            @pl.when(i != 3 * nsteps)
            def _():
                s_ref[...] += ps
                sq_ref[...] += psq

    @pl.when(i % 2 == 0)
    def _():
        dma_wait(i, buf0, sem0, sem0b)
        compute(buf0)

    @pl.when(i % 2 == 1)
    def _():
        dma_wait(i, buf1, sem1, sem1b)
        compute(buf1)


def _main(adj1, adj2, x, wu1, wu2, ba, bb, wa, wb, b2, b4, b6, v3, c3, bm):
    n, fin = x.shape
    h2 = wu1.shape[1]
    d2 = wa.shape[1]
    d = d2 // 2
    nsteps = n // bm
    const = lambda i: (0, 0)
    p3row = lambda i: (jnp.where(i // nsteps == 3, i % nsteps, 0), 0)
    vspec = lambda shape: pl.BlockSpec(shape, const)
    return pl.pallas_call(
        functools.partial(_main_body, bm=bm, nsteps=nsteps),
        grid=(4 * nsteps,),
        in_specs=[vspec((n, fin)), vspec((fin, h2)), vspec((fin, h2)),
                  vspec((1, h2)), vspec((1, h2)),
                  vspec((h2, d2)), vspec((h2, d2)),
                  vspec((1, d)), vspec((1, d)), vspec((1, d)),
                  vspec((3 * d, 3)), vspec((1, 3)),
                  pl.BlockSpec(memory_space=pltpu.MemorySpace.HBM),
                  pl.BlockSpec(memory_space=pltpu.MemorySpace.HBM)],
        out_specs=[pl.BlockSpec((bm, d), p3row)] * 5
        + [pl.BlockSpec((1, d), const), pl.BlockSpec((1, d), const)],
        out_shape=[jax.ShapeDtypeStruct((n, d), jnp.float32)] * 5
        + [jax.ShapeDtypeStruct((1, d), jnp.float32)] * 2,
        scratch_shapes=[pltpu.VMEM((bm, n), jnp.float32),
                        pltpu.VMEM((bm, n), jnp.float32),
                        pltpu.VMEM((n, 2 * h2), jnp.float32),
                        pltpu.VMEM((n, 4 * d), jnp.float32),
                        pltpu.VMEM((n, 3 * d), jnp.float32),
                        pltpu.SemaphoreType.DMA((2,)),
                        pltpu.SemaphoreType.DMA((2,)),
                        pltpu.SemaphoreType.DMA((2,)),
                        pltpu.SemaphoreType.DMA((2,))],
        compiler_params=_VMEM,
    )(x, wu1, wu2, ba, bb, wa, wb, b2, b4, b6, v3, c3, adj1, adj2)


def _bnorm_body(emb_ref, s_ref, sq_ref, g_ref, beta_ref, lwt_ref, lb_ref,
                embn_ref, lp_ref, *, inv_n):
    mu = s_ref[...] * inv_n
    var = sq_ref[...] * inv_n - mu * mu
    emb = emb_ref[...]
    embn = (emb - mu) / jnp.sqrt(var + 1e-5) * g_ref[...] + beta_ref[...]
    out = jnp.dot(embn, lwt_ref[...],
                  preferred_element_type=jnp.float32) + lb_ref[...]
    mo = jnp.max(out, axis=1, keepdims=True)
    lse = mo + jnp.log(jnp.sum(jnp.exp(out - mo), axis=1, keepdims=True))
    embn_ref[...] = embn
    lp_ref[...] = out - lse


def _bnorm(emb, s, sq, g, beta, lwt, lb, bm):
    n, d = emb.shape
    c = lwt.shape[1]
    row = lambda i: (i, 0)
    const = lambda i: (0, 0)
    return pl.pallas_call(
        functools.partial(_bnorm_body, inv_n=1.0 / n),
        grid=(n // bm,),
        in_specs=[pl.BlockSpec((bm, d), row),
                  pl.BlockSpec((1, d), const),
                  pl.BlockSpec((1, d), const),
                  pl.BlockSpec((1, d), const),
                  pl.BlockSpec((1, d), const),
                  pl.BlockSpec((d, c), const),
                  pl.BlockSpec((1, c), const)],
        out_specs=[pl.BlockSpec((bm, d), row), pl.BlockSpec((bm, c), row)],
        out_shape=[jax.ShapeDtypeStruct((n, d), jnp.float32),
                   jax.ShapeDtypeStruct((n, c), jnp.float32)],
        compiler_params=_VMEM,
    )(emb, s, sq, g, beta, lwt, lb)


def kernel(x, adj1, adj2, gc1_w, gc1_b, gc2_w, gc2_b, gc3_w, gc3_b,
           gc4_w, gc4_b, gc5_w, gc5_b, gc6_w, gc6_b, W1, b1, W2, b2,
           W3, b3, Q, lin_w, lin_b, bn_gamma, bn_beta):
    n = x.shape[0]
    h = gc1_w.shape[1]
    d = gc2_w.shape[1]

    wu1 = jnp.concatenate([gc1_w, gc5_w], axis=1)
    wu2 = jnp.concatenate([gc3_w, gc5_w], axis=1)
    ba = jnp.concatenate([gc1_b, gc5_b])[None, :]
    bb = jnp.concatenate([gc3_b, gc5_b])[None, :]
    zh = jnp.zeros((h, d), jnp.float32)
    wa = jnp.concatenate(
        [jnp.concatenate([gc2_w, zh], axis=1),
         jnp.concatenate([zh, gc6_w], axis=1)], axis=0)
    wb = jnp.concatenate(
        [jnp.concatenate([gc4_w, zh], axis=1),
         jnp.concatenate([zh, gc6_w], axis=1)], axis=0)
    zd = jnp.zeros((d, 1), jnp.float32)
    v3 = jnp.concatenate(
        [jnp.concatenate([W1 @ Q, zd, zd], axis=1),
         jnp.concatenate([zd, W2 @ Q, zd], axis=1),
         jnp.concatenate([zd, zd, W3 @ Q], axis=1)], axis=0)
    c3 = jnp.concatenate([b1 @ Q, b2 @ Q, b3 @ Q])[None, :]

    bm_big = 400 if n % 400 == 0 else n
    bm_small = 1000 if n % 1000 == 0 else n

    x1, x2, x1c, x2c, emb, s, sq = _main(
        adj1, adj2, x, wu1, wu2, ba, bb, wa, wb,
        gc2_b[None, :], gc4_b[None, :], gc6_b[None, :], v3, c3, bm_big)
    embn, lp = _bnorm(emb, s, sq, bn_gamma[None, :], bn_beta[None, :],
                      lin_w.T, lin_b[None, :], bm_small)
    return (x1, x2, x1c, x2c, embn, lp)


# speedup vs baseline: 1.0006x; 1.0006x over previous
import functools

import jax
import jax.numpy as jnp
from jax.experimental import pallas as pl
from jax.experimental.pallas import tpu as pltpu

_VMEM = pltpu.CompilerParams(vmem_limit_bytes=100 * 1024 * 1024)


def _lrelu(v):
    return jnp.where(v >= 0, v, 0.2 * v)


def _main_body(x_ref, wu1_ref, wu2_ref, ba_ref, bb_ref, wa_ref, wb_ref,
               b2_ref, b4_ref, b6_ref, v3_ref, c3_ref, adj1_ref, adj2_ref,
               x1_ref, x2_ref, x1c_ref, x2c_ref, emb_ref, s_ref, sq_ref,
               buf0, buf1, u_s, t_s, z_s, sem0, sem1, sem0b, sem1b,
               *, bm, nsteps):
    n = x_ref.shape[0]
    h2 = wu1_ref.shape[1]
    d = b2_ref.shape[1]
    i = pl.program_id(0)
    total = 4 * nsteps

    hm = (bm // 16) * 8
    hm2 = bm - hm

    def _dma(k, bufref, semref, semref2, *, wait):
        pk = k // nsteps
        rk = (k % nsteps) * bm

        def act(src):
            c1 = pltpu.make_async_copy(src.at[pl.ds(rk, hm), :],
                                       bufref.at[pl.ds(0, hm), :], semref)
            c2 = pltpu.make_async_copy(src.at[pl.ds(rk + hm, hm2), :],
                                       bufref.at[pl.ds(hm, hm2), :], semref2)
            if wait:
                c1.wait()
                c2.wait()
            else:
                c1.start()
                c2.start()

        @pl.when((pk == 0) | (pk == 2))
        def _():
            act(adj1_ref)

        @pl.when((pk == 1) | (pk == 3))
        def _():
            act(adj2_ref)

    def dma_start(k, bufref, semref, semref2):
        _dma(k, bufref, semref, semref2, wait=False)

    def dma_wait(k, bufref, semref, semref2):
        _dma(k, bufref, semref, semref2, wait=True)

    @pl.when(i == 0)
    def _():
        dma_start(0, buf0, sem0, sem0b)
        nchunk = 10 if n % 10 == 0 else 1
        cs = n // nchunk
        for c in range(nchunk):
            xc_v = x_ref[pl.ds(c * cs, cs), :]
            u_s[pl.ds(c * cs, cs), 0:h2] = jnp.dot(
                xc_v, wu1_ref[...], preferred_element_type=jnp.float32)
            u_s[pl.ds(c * cs, cs), h2:2 * h2] = jnp.dot(
                xc_v, wu2_ref[...], preferred_element_type=jnp.float32)

    nxt = i + 1

    @pl.when((nxt < total) & (nxt % 2 == 0))
    def _():
        dma_start(nxt, buf0, sem0, sem0b)

    @pl.when((nxt < total) & (nxt % 2 == 1))
    def _():
        dma_start(nxt, buf1, sem1, sem1b)

    p = i // nsteps
    r = (i % nsteps) * bm

    def compute(aref):
        @pl.when(p == 0)
        def _():
            y = jnp.dot(aref[...], u_s[:, 0:h2],
                        preferred_element_type=jnp.float32)
            t_s[pl.ds(r, bm), 0:2 * d] = jnp.dot(
                _lrelu(y + ba_ref[...]), wa_ref[...],
                preferred_element_type=jnp.float32)

        @pl.when(p == 1)
        def _():
            y = jnp.dot(aref[...], u_s[:, h2:2 * h2],
                        preferred_element_type=jnp.float32)
            t_s[pl.ds(r, bm), 2 * d:4 * d] = jnp.dot(
                _lrelu(y + bb_ref[...]), wb_ref[...],
                preferred_element_type=jnp.float32)

        @pl.when(p == 2)
        def _():
            z_s[pl.ds(r, bm), 0:2 * d] = jnp.dot(
                aref[...], t_s[:, 0:2 * d],
                preferred_element_type=jnp.float32)
            z_s[pl.ds(r, bm), 2 * d:3 * d] = jnp.dot(
                aref[...], t_s[:, 3 * d:4 * d],
                preferred_element_type=jnp.float32)

        @pl.when(p == 3)
        def _():
            z2 = jnp.dot(aref[...], t_s[:, 2 * d:3 * d],
                         preferred_element_type=jnp.float32)
            za = z_s[pl.ds(r, bm), 0:2 * d]
            x1 = za[:, 0:d] + b2_ref[...]
            x1c = za[:, d:2 * d] + b6_ref[...]
            x2c = z_s[pl.ds(r, bm), 2 * d:3 * d] + b6_ref[...]
            x2 = z2 + b4_ref[...]
            xc = (x1c + x2c) * 0.5

            x3 = jnp.concatenate([x1, x2, xc], axis=1)
            s = jnp.dot(x3, v3_ref[...], preferred_element_type=jnp.float32)
            s = _lrelu(s + c3_ref[...])
            m = jnp.max(s, axis=1, keepdims=True)
            e = jnp.exp(s - m)
            w = e / jnp.sum(e, axis=1, keepdims=True)
            emb = w[:, 0:1] * x1 + w[:, 1:2] * x2 + w[:, 2:3] * xc

            x1_ref[...] = x1
            x2_ref[...] = x2
            x1c_ref[...] = x1c
            x2c_ref[...] = x2c
            emb_ref[...] = emb
            ps = jnp.sum(emb, axis=0, keepdims=True)
            psq = jnp.sum(emb * emb, axis=0, keepdims=True)

            @pl.when(i == 3 * nsteps)
            def _():
                s_ref[...] = ps
                sq_ref[...] = psq

            @pl.when(i != 3 * nsteps)
            def _():
                s_ref[...] += ps
                sq_ref[...] += psq

    @pl.when(i % 2 == 0)
    def _():
        dma_wait(i, buf0, sem0, sem0b)
        compute(buf0)

    @pl.when(i % 2 == 1)
    def _():
        dma_wait(i, buf1, sem1, sem1b)
        compute(buf1)


def _main(adj1, adj2, x, wu1, wu2, ba, bb, wa, wb, b2, b4, b6, v3, c3, bm):
    n, fin = x.shape
    h2 = wu1.shape[1]
    d2 = wa.shape[1]
    d = d2 // 2
    nsteps = n // bm
    const = lambda i: (0, 0)
    p3row = lambda i: (jnp.where(i // nsteps == 3, i % nsteps, 0), 0)
    vspec = lambda shape: pl.BlockSpec(shape, const)
    return pl.pallas_call(
        functools.partial(_main_body, bm=bm, nsteps=nsteps),
        grid=(4 * nsteps,),
        in_specs=[vspec((n, fin)), vspec((fin, h2)), vspec((fin, h2)),
                  vspec((1, h2)), vspec((1, h2)),
                  vspec((h2, d2)), vspec((h2, d2)),
                  vspec((1, d)), vspec((1, d)), vspec((1, d)),
                  vspec((3 * d, 3)), vspec((1, 3)),
                  pl.BlockSpec(memory_space=pltpu.MemorySpace.HBM),
                  pl.BlockSpec(memory_space=pltpu.MemorySpace.HBM)],
        out_specs=[pl.BlockSpec((bm, d), p3row)] * 5
        + [pl.BlockSpec((1, d), const), pl.BlockSpec((1, d), const)],
        out_shape=[jax.ShapeDtypeStruct((n, d), jnp.float32)] * 5
        + [jax.ShapeDtypeStruct((1, d), jnp.float32)] * 2,
        scratch_shapes=[pltpu.VMEM((bm, n), jnp.float32),
                        pltpu.VMEM((bm, n), jnp.float32),
                        pltpu.VMEM((n, 2 * h2), jnp.float32),
                        pltpu.VMEM((n, 4 * d), jnp.float32),
                        pltpu.VMEM((n, 3 * d), jnp.float32),
                        pltpu.SemaphoreType.DMA,
                        pltpu.SemaphoreType.DMA,
                        pltpu.SemaphoreType.DMA,
                        pltpu.SemaphoreType.DMA],
        compiler_params=_VMEM,
    )(x, wu1, wu2, ba, bb, wa, wb, b2, b4, b6, v3, c3, adj1, adj2)


def _bnorm_body(emb_ref, s_ref, sq_ref, g_ref, beta_ref, lwt_ref, lb_ref,
                embn_ref, lp_ref, *, inv_n):
    mu = s_ref[...] * inv_n
    var = sq_ref[...] * inv_n - mu * mu
    emb = emb_ref[...]
    embn = (emb - mu) / jnp.sqrt(var + 1e-5) * g_ref[...] + beta_ref[...]
    out = jnp.dot(embn, lwt_ref[...],
                  preferred_element_type=jnp.float32) + lb_ref[...]
    mo = jnp.max(out, axis=1, keepdims=True)
    lse = mo + jnp.log(jnp.sum(jnp.exp(out - mo), axis=1, keepdims=True))
    embn_ref[...] = embn
    lp_ref[...] = out - lse


def _bnorm(emb, s, sq, g, beta, lwt, lb, bm):
    n, d = emb.shape
    c = lwt.shape[1]
    row = lambda i: (i, 0)
    const = lambda i: (0, 0)
    return pl.pallas_call(
        functools.partial(_bnorm_body, inv_n=1.0 / n),
        grid=(n // bm,),
        in_specs=[pl.BlockSpec((bm, d), row),
                  pl.BlockSpec((1, d), const),
                  pl.BlockSpec((1, d), const),
                  pl.BlockSpec((1, d), const),
                  pl.BlockSpec((1, d), const),
                  pl.BlockSpec((d, c), const),
                  pl.BlockSpec((1, c), const)],
        out_specs=[pl.BlockSpec((bm, d), row), pl.BlockSpec((bm, c), row)],
        out_shape=[jax.ShapeDtypeStruct((n, d), jnp.float32),
                   jax.ShapeDtypeStruct((n, c), jnp.float32)],
        compiler_params=_VMEM,
    )(emb, s, sq, g, beta, lwt, lb)


def kernel(x, adj1, adj2, gc1_w, gc1_b, gc2_w, gc2_b, gc3_w, gc3_b,
           gc4_w, gc4_b, gc5_w, gc5_b, gc6_w, gc6_b, W1, b1, W2, b2,
           W3, b3, Q, lin_w, lin_b, bn_gamma, bn_beta):
    n = x.shape[0]
    h = gc1_w.shape[1]
    d = gc2_w.shape[1]

    wu1 = jnp.concatenate([gc1_w, gc5_w], axis=1)
    wu2 = jnp.concatenate([gc3_w, gc5_w], axis=1)
    ba = jnp.concatenate([gc1_b, gc5_b])[None, :]
    bb = jnp.concatenate([gc3_b, gc5_b])[None, :]
    zh = jnp.zeros((h, d), jnp.float32)
    wa = jnp.concatenate(
        [jnp.concatenate([gc2_w, zh], axis=1),
         jnp.concatenate([zh, gc6_w], axis=1)], axis=0)
    wb = jnp.concatenate(
        [jnp.concatenate([gc4_w, zh], axis=1),
         jnp.concatenate([zh, gc6_w], axis=1)], axis=0)
    zd = jnp.zeros((d, 1), jnp.float32)
    v3 = jnp.concatenate(
        [jnp.concatenate([W1 @ Q, zd, zd], axis=1),
         jnp.concatenate([zd, W2 @ Q, zd], axis=1),
         jnp.concatenate([zd, zd, W3 @ Q], axis=1)], axis=0)
    c3 = jnp.concatenate([b1 @ Q, b2 @ Q, b3 @ Q])[None, :]

    bm_big = 400 if n % 400 == 0 else n
    bm_small = 1000 if n % 1000 == 0 else n

    x1, x2, x1c, x2c, emb, s, sq = _main(
        adj1, adj2, x, wu1, wu2, ba, bb, wa, wb,
        gc2_b[None, :], gc4_b[None, :], gc6_b[None, :], v3, c3, bm_big)
    embn, lp = _bnorm(emb, s, sq, bn_gamma[None, :], bn_beta[None, :],
                      lin_w.T, lin_b[None, :], bm_small)
    return (x1, x2, x1c, x2c, embn, lp)


# speedup vs baseline: 1.0083x; 1.0076x over previous
import functools

import jax
import jax.numpy as jnp
from jax.experimental import pallas as pl
from jax.experimental.pallas import tpu as pltpu

_VMEM = pltpu.CompilerParams(vmem_limit_bytes=100 * 1024 * 1024)


def _lrelu(v):
    return jnp.where(v >= 0, v, 0.2 * v)


def _main_body(x_ref, wu1_ref, wu2_ref, ba_ref, bb_ref, wa_ref, wb_ref,
               b2_ref, b4_ref, b6_ref, v3_ref, c3_ref, adj1_ref, adj2_ref,
               x1_ref, x2_ref, x1c_ref, x2c_ref, emb_ref, s_ref, sq_ref,
               buf0, buf1, u_s, t_s, z_s, sem0, sem1, sem0b, sem1b,
               *, bm, nsteps):
    n = x_ref.shape[0]
    h2 = wu1_ref.shape[1]
    d = b2_ref.shape[1]
    i = pl.program_id(0)
    total = 4 * nsteps

    hm = (bm // 16) * 8
    hm2 = bm - hm

    def _dma(k, bufref, semref, semref2, *, wait):
        pk = k // nsteps
        rk = (k % nsteps) * bm

        def act(src):
            c1 = pltpu.make_async_copy(src.at[pl.ds(rk, hm), :],
                                       bufref.at[pl.ds(0, hm), :], semref)
            c2 = pltpu.make_async_copy(src.at[pl.ds(rk + hm, hm2), :],
                                       bufref.at[pl.ds(hm, hm2), :], semref2)
            if wait:
                c1.wait()
                c2.wait()
            else:
                c1.start()
                c2.start()

        @pl.when((pk == 0) | (pk == 2))
        def _():
            act(adj1_ref)

        @pl.when((pk == 1) | (pk == 3))
        def _():
            act(adj2_ref)

    def dma_start(k, bufref, semref, semref2):
        _dma(k, bufref, semref, semref2, wait=False)

    def dma_wait(k, bufref, semref, semref2):
        _dma(k, bufref, semref, semref2, wait=True)

    @pl.when(i == 0)
    def _():
        dma_start(0, buf0, sem0, sem0b)
        nchunk = 10 if n % 10 == 0 else 1
        cs = n // nchunk
        for c in range(nchunk):
            xc_v = x_ref[pl.ds(c * cs, cs), :]
            u_s[pl.ds(c * cs, cs), 0:h2] = jnp.dot(
                xc_v, wu1_ref[...], preferred_element_type=jnp.float32)
            u_s[pl.ds(c * cs, cs), h2:2 * h2] = jnp.dot(
                xc_v, wu2_ref[...], preferred_element_type=jnp.float32)

    nxt = i + 1

    @pl.when((nxt < total) & (nxt % 2 == 0))
    def _():
        dma_start(nxt, buf0, sem0, sem0b)

    @pl.when((nxt < total) & (nxt % 2 == 1))
    def _():
        dma_start(nxt, buf1, sem1, sem1b)

    p = i // nsteps
    r = (i % nsteps) * bm

    def compute(aref):
        @pl.when(p == 0)
        def _():
            y = jnp.dot(aref[...], u_s[:, 0:h2],
                        preferred_element_type=jnp.float32)
            t_s[pl.ds(r, bm), 0:2 * d] = jnp.dot(
                _lrelu(y + ba_ref[...]), wa_ref[...],
                preferred_element_type=jnp.float32)

        @pl.when(p == 1)
        def _():
            y = jnp.dot(aref[...], u_s[:, h2:2 * h2],
                        preferred_element_type=jnp.float32)
            t_s[pl.ds(r, bm), 2 * d:4 * d] = jnp.dot(
                _lrelu(y + bb_ref[...]), wb_ref[...],
                preferred_element_type=jnp.float32)

        @pl.when(p == 2)
        def _():
            z_s[pl.ds(r, bm), 0:3 * d] = jnp.dot(
                aref[...], t_s[:, 0:3 * d],
                preferred_element_type=jnp.float32)

        @pl.when(p == 3)
        def _():
            z2 = jnp.dot(aref[...], t_s[:, 3 * d:4 * d],
                         preferred_element_type=jnp.float32)
            za = z_s[pl.ds(r, bm), 0:2 * d]
            x1 = za[:, 0:d] + b2_ref[...]
            x1c = za[:, d:2 * d] + b6_ref[...]
            x2c = z_s[pl.ds(r, bm), 2 * d:3 * d] + b6_ref[...]
            x2 = z2 + b4_ref[...]
            xc = (x1c + x2c) * 0.5

            x3 = jnp.concatenate([x1, x2, xc], axis=1)
            s = jnp.dot(x3, v3_ref[...], preferred_element_type=jnp.float32)
            s = _lrelu(s + c3_ref[...])
            m = jnp.max(s, axis=1, keepdims=True)
            e = jnp.exp(s - m)
            w = e / jnp.sum(e, axis=1, keepdims=True)
            emb = w[:, 0:1] * x1 + w[:, 1:2] * x2 + w[:, 2:3] * xc

            x1_ref[...] = x1
            x2_ref[...] = x2
            x1c_ref[...] = x1c
            x2c_ref[...] = x2c
            emb_ref[...] = emb
            ps = jnp.sum(emb, axis=0, keepdims=True)
            psq = jnp.sum(emb * emb, axis=0, keepdims=True)

            @pl.when(i == 3 * nsteps)
            def _():
                s_ref[...] = ps
                sq_ref[...] = psq

            @pl.when(i != 3 * nsteps)
            def _():
                s_ref[...] += ps
                sq_ref[...] += psq

    @pl.when(i % 2 == 0)
    def _():
        dma_wait(i, buf0, sem0, sem0b)
        compute(buf0)

    @pl.when(i % 2 == 1)
    def _():
        dma_wait(i, buf1, sem1, sem1b)
        compute(buf1)


def _main(adj1, adj2, x, wu1, wu2, ba, bb, wa, wb, b2, b4, b6, v3, c3, bm):
    n, fin = x.shape
    h2 = wu1.shape[1]
    d2 = wa.shape[1]
    d = d2 // 2
    nsteps = n // bm
    const = lambda i: (0, 0)
    p3row = lambda i: (jnp.where(i // nsteps == 3, i % nsteps, 0), 0)
    vspec = lambda shape: pl.BlockSpec(shape, const)
    return pl.pallas_call(
        functools.partial(_main_body, bm=bm, nsteps=nsteps),
        grid=(4 * nsteps,),
        in_specs=[vspec((n, fin)), vspec((fin, h2)), vspec((fin, h2)),
                  vspec((1, h2)), vspec((1, h2)),
                  vspec((h2, d2)), vspec((h2, d2)),
                  vspec((1, d)), vspec((1, d)), vspec((1, d)),
                  vspec((3 * d, 3)), vspec((1, 3)),
                  pl.BlockSpec(memory_space=pltpu.MemorySpace.HBM),
                  pl.BlockSpec(memory_space=pltpu.MemorySpace.HBM)],
        out_specs=[pl.BlockSpec((bm, d), p3row)] * 5
        + [pl.BlockSpec((1, d), const), pl.BlockSpec((1, d), const)],
        out_shape=[jax.ShapeDtypeStruct((n, d), jnp.float32)] * 5
        + [jax.ShapeDtypeStruct((1, d), jnp.float32)] * 2,
        scratch_shapes=[pltpu.VMEM((bm, n), jnp.float32),
                        pltpu.VMEM((bm, n), jnp.float32),
                        pltpu.VMEM((n, 2 * h2), jnp.float32),
                        pltpu.VMEM((n, 4 * d), jnp.float32),
                        pltpu.VMEM((n, 3 * d), jnp.float32),
                        pltpu.SemaphoreType.DMA,
                        pltpu.SemaphoreType.DMA,
                        pltpu.SemaphoreType.DMA,
                        pltpu.SemaphoreType.DMA],
        compiler_params=_VMEM,
    )(x, wu1, wu2, ba, bb, wa, wb, b2, b4, b6, v3, c3, adj1, adj2)


def _bnorm_body(emb_ref, s_ref, sq_ref, g_ref, beta_ref, lwt_ref, lb_ref,
                embn_ref, lp_ref, *, inv_n):
    mu = s_ref[...] * inv_n
    var = sq_ref[...] * inv_n - mu * mu
    emb = emb_ref[...]
    embn = (emb - mu) / jnp.sqrt(var + 1e-5) * g_ref[...] + beta_ref[...]
    out = jnp.dot(embn, lwt_ref[...],
                  preferred_element_type=jnp.float32) + lb_ref[...]
    mo = jnp.max(out, axis=1, keepdims=True)
    lse = mo + jnp.log(jnp.sum(jnp.exp(out - mo), axis=1, keepdims=True))
    embn_ref[...] = embn
    lp_ref[...] = out - lse


def _bnorm(emb, s, sq, g, beta, lwt, lb, bm):
    n, d = emb.shape
    c = lwt.shape[1]
    row = lambda i: (i, 0)
    const = lambda i: (0, 0)
    return pl.pallas_call(
        functools.partial(_bnorm_body, inv_n=1.0 / n),
        grid=(n // bm,),
        in_specs=[pl.BlockSpec((bm, d), row),
                  pl.BlockSpec((1, d), const),
                  pl.BlockSpec((1, d), const),
                  pl.BlockSpec((1, d), const),
                  pl.BlockSpec((1, d), const),
                  pl.BlockSpec((d, c), const),
                  pl.BlockSpec((1, c), const)],
        out_specs=[pl.BlockSpec((bm, d), row), pl.BlockSpec((bm, c), row)],
        out_shape=[jax.ShapeDtypeStruct((n, d), jnp.float32),
                   jax.ShapeDtypeStruct((n, c), jnp.float32)],
        compiler_params=_VMEM,
    )(emb, s, sq, g, beta, lwt, lb)


def kernel(x, adj1, adj2, gc1_w, gc1_b, gc2_w, gc2_b, gc3_w, gc3_b,
           gc4_w, gc4_b, gc5_w, gc5_b, gc6_w, gc6_b, W1, b1, W2, b2,
           W3, b3, Q, lin_w, lin_b, bn_gamma, bn_beta):
    n = x.shape[0]
    h = gc1_w.shape[1]
    d = gc2_w.shape[1]

    wu1 = jnp.concatenate([gc1_w, gc5_w], axis=1)
    wu2 = jnp.concatenate([gc3_w, gc5_w], axis=1)
    ba = jnp.concatenate([gc1_b, gc5_b])[None, :]
    bb = jnp.concatenate([gc3_b, gc5_b])[None, :]
    zh = jnp.zeros((h, d), jnp.float32)
    wa = jnp.concatenate(
        [jnp.concatenate([gc2_w, zh], axis=1),
         jnp.concatenate([zh, gc6_w], axis=1)], axis=0)
    wb = jnp.concatenate(
        [jnp.concatenate([zh, gc4_w], axis=1),
         jnp.concatenate([gc6_w, zh], axis=1)], axis=0)
    zd = jnp.zeros((d, 1), jnp.float32)
    v3 = jnp.concatenate(
        [jnp.concatenate([W1 @ Q, zd, zd], axis=1),
         jnp.concatenate([zd, W2 @ Q, zd], axis=1),
         jnp.concatenate([zd, zd, W3 @ Q], axis=1)], axis=0)
    c3 = jnp.concatenate([b1 @ Q, b2 @ Q, b3 @ Q])[None, :]

    bm_big = 400 if n % 400 == 0 else n
    bm_small = 1000 if n % 1000 == 0 else n

    x1, x2, x1c, x2c, emb, s, sq = _main(
        adj1, adj2, x, wu1, wu2, ba, bb, wa, wb,
        gc2_b[None, :], gc4_b[None, :], gc6_b[None, :], v3, c3, bm_big)
    embn, lp = _bnorm(emb, s, sq, bn_gamma[None, :], bn_beta[None, :],
                      lin_w.T, lin_b[None, :], bm_small)
    return (x1, x2, x1c, x2c, embn, lp)
